# baseline (speedup 1.0000x reference)
import jax
import jax.numpy as jnp
from jax import lax
from jax.experimental import pallas as pl
from jax.experimental.pallas import tpu as pltpu
from jax.experimental.pallas import tpu_sc as plsc

BATCH = 16384
EMBED = 64
NC = 2
NS = 16
NW = NC * NS
B_PER_W = BATCH // NW
CHUNK = 128
NCHUNK = B_PER_W // CHUNK
RING = 10
CH2 = 128
NCH2 = B_PER_W // CH2
RING2 = 2
SENTINEL = 0x7FFFFF


def _k1_user(su_hbm, ut3, wb_hbm, u1d_hbm, suv, colv, ucols, sbuf, ustage,
             wv, sems):
    wid = lax.axis_index("s") * NC + lax.axis_index("c")
    base = wid * B_PER_W

    pltpu.sync_copy(su_hbm.at[pl.ds(base, B_PER_W)], suv)
    pltpu.sync_copy(wb_hbm, wv)

    iot = lax.iota(jnp.int32, 16)

    for k in range(B_PER_W // 16):
        colv[pl.ds(k * 16, 16)] = jax.lax.shift_right_logical(
            suv[pl.ds(k * 16, 16)], 7)
    colv[pl.ds(B_PER_W, 16)] = jnp.full((16,), SENTINEL, jnp.int32)

    nu = jnp.int32(0)
    for k in range(B_PER_W // 16):
        cur = colv[pl.ds(k * 16, 16)]
        prev = plsc.load_gather(colv, [jnp.maximum(iot + (k * 16 - 1), 0)])
        m = cur != prev
        if k == 0:
            m = jnp.logical_or(m, iot == 0)
        ranks = plsc.cumsum(m.astype(jnp.int32))
        plsc.store_scatter(ucols, [ranks + (nu - 1)], cur, mask=m)
        nu = nu + ranks[15]

    wchunks = [wv[pl.ds(q * 16, 16)] for q in range(EMBED // 16)]
    ehq = [(iot + 16 * q) >> 3 for q in range(4)]
    elq = [(iot + 16 * q) & 7 for q in range(4)]
    eoffq = [iot + 16 * q for q in range(4)]

    def ucol(m):
        return plsc.load_gather(ucols, [jnp.full((16,), m, jnp.int32)])[0]

    def fire(m, b):
        @pl.when(m < nu)
        def _():
            c = ucol(m)
            pltpu.async_copy(
                ut3.at[:, :, pl.ds(c * CHUNK, CHUNK)], sbuf.at[b], sems.at[b])

    def drain(m, b):
        @pl.when(m < nu)
        def _():
            pltpu.make_async_copy(
                ut3.at[:, :, pl.ds(0, CHUNK)], sbuf.at[b], sems.at[b]).wait()

    def extract(m, b, carry):
        c = ucol(m)
        def cond(carry):
            jp, col_j = carry
            return jnp.logical_and(jp < B_PER_W, col_j == c)

        def wbody(carry):
            jp, _ = carry
            jpv = jnp.full((16,), jp, jnp.int32)
            su_j = plsc.load_gather(suv, [jpv])
            liv = su_j & 127
            fbase = jpv << 6
            for q in range(4):
                val = plsc.load_gather(sbuf, [jnp.full((16,), b, jnp.int32),
                                              ehq[q], elq[q], liv])
                flat = fbase + eoffq[q]
                plsc.store_scatter(
                    ustage, [jax.lax.shift_right_logical(flat, 7), flat & 127],
                    val * wchunks[q])
            jp1 = jp + 1
            col1 = plsc.load_gather(colv, [jnp.full((16,), jp1, jnp.int32)])[0]
            return (jp1, col1)

        return lax.while_loop(cond, wbody, carry)

    for r in range(RING):
        fire(jnp.int32(r), r)
    nsteps = (nu + RING - 1) // RING

    def ring_body(t, carry):
        m0 = RING * t
        for r in range(RING):
            drain(m0 + r, r)
            carry = extract(m0 + r, r, carry)
            fire(m0 + r + RING, r)
        return carry

    col0 = colv[pl.ds(0, 16)][0]
    lax.fori_loop(0, nsteps, ring_body, (jnp.int32(0), col0))

    pltpu.sync_copy(ustage, u1d_hbm.at[pl.ds(wid * (B_PER_W // 2),
                                             B_PER_W // 2)])


def _k2_join(u1d_hbm, inv_hbm, pids_hbm, product_t2, wb_hbm, out_hbm,
             iidx, pidx, iphys, pphys, ubuf, pbuf, wv, outv, sems):
    wid = lax.axis_index("s") * NC + lax.axis_index("c")
    base = wid * B_PER_W

    pltpu.sync_copy(inv_hbm.at[pl.ds(base, B_PER_W)], iidx)
    pltpu.sync_copy(pids_hbm.at[pl.ds(base, B_PER_W)], pidx)
    pltpu.sync_copy(wb_hbm, wv)

    for k in range(B_PER_W // 16):
        sl = pl.ds(k * 16, 16)
        iphys[sl] = jax.lax.shift_right_logical(iidx[sl], 1)
        pphys[sl] = jax.lax.shift_right_logical(pidx[sl], 1)

    def fire(j):
        b = j % RING2
        return (
            pltpu.async_copy(
                u1d_hbm.at[iphys.at[pl.ds(j * CH2, CH2)]], ubuf.at[b],
                sems.at[b]),
            pltpu.async_copy(
                product_t2.at[pphys.at[pl.ds(j * CH2, CH2)]], pbuf.at[b],
                sems.at[RING2 + b]),
        )

    lane = lax.iota(jnp.int32, 16)
    bias = wv[pl.ds(EMBED, 16)][0]

    inflight = {0: fire(0)}
    for j in range(NCH2):
        if j + 1 < NCH2:
            inflight[j + 1] = fire(j + 1)
        for c in inflight.pop(j):
            c.wait()
        urows = ubuf.at[j % RING2]
        prows = pbuf.at[j % RING2]

        def body(g, carry):
            k = j * CH2 + g * 16
            rows = lane + g * 16
            ucol0 = (iidx[pl.ds(k, 16)] & 1) * EMBED
            pcol0 = (pidx[pl.ds(k, 16)] & 1) * EMBED
            acc = jnp.zeros((16,), jnp.float32)
            for e in range(EMBED):
                u = plsc.load_gather(urows, [rows, ucol0 + e])
                p = plsc.load_gather(prows, [rows, pcol0 + e])
                acc = acc + u * p
            outv[pl.ds(k, 16)] = acc + bias
            return carry

        lax.fori_loop(0, CH2 // 16, body, 0)

    pltpu.sync_copy(outv, out_hbm.at[pl.ds(base, B_PER_W)])


@jax.jit
def kernel(user_ids, product_ids, user_table, product_table, fc_w, fc_b):
    uids = user_ids.astype(jnp.int32)
    pids = product_ids.astype(jnp.int32)
    iota = lax.iota(jnp.int32, BATCH)
    su, order = lax.sort_key_val(uids, iota)
    _, inv = lax.sort_key_val(order, iota)

    ut3 = user_table.T.reshape(8, 8, user_table.shape[0])
    product_t2 = product_table.reshape(product_table.shape[0] // 2, 2 * EMBED)

    wb = jnp.zeros((128,), jnp.float32)
    wb = wb.at[:EMBED].set(fc_w[0]).at[EMBED].set(fc_b[0])

    mesh = plsc.VectorSubcoreMesh(core_axis_name="c", subcore_axis_name="s")
    params = pltpu.CompilerParams(needs_layout_passes=False)

    k1 = pl.kernel(
        _k1_user,
        out_type=jax.ShapeDtypeStruct((BATCH // 2, 2 * EMBED), jnp.float32),
        mesh=mesh, compiler_params=params,
        scratch_types=[
            pltpu.VMEM((B_PER_W,), jnp.int32),
            pltpu.VMEM((B_PER_W + 16,), jnp.int32),
            pltpu.VMEM((B_PER_W,), jnp.int32),
            pltpu.VMEM((RING, 8, 8, CHUNK), jnp.float32),
            pltpu.VMEM((B_PER_W // 2, 2 * EMBED), jnp.float32),
            pltpu.VMEM((128,), jnp.float32),
            pltpu.SemaphoreType.DMA((RING,)),
        ],
    )
    u1d = k1(su, ut3, wb)

    k2 = pl.kernel(
        _k2_join,
        out_type=jax.ShapeDtypeStruct((BATCH,), jnp.float32),
        mesh=mesh, compiler_params=params,
        scratch_types=[
            pltpu.VMEM((B_PER_W,), jnp.int32),
            pltpu.VMEM((B_PER_W,), jnp.int32),
            pltpu.VMEM((B_PER_W,), jnp.int32),
            pltpu.VMEM((B_PER_W,), jnp.int32),
            pltpu.VMEM((RING2, CH2, 2 * EMBED), jnp.float32),
            pltpu.VMEM((RING2, CH2, 2 * EMBED), jnp.float32),
            pltpu.VMEM((128,), jnp.float32),
            pltpu.VMEM((B_PER_W,), jnp.float32),
            pltpu.SemaphoreType.DMA((RING2,)),
        ],
    )
    return k2(u1d, inv, pids, product_t2, wb)


# speedup vs baseline: 1.0106x; 1.0106x over previous
import jax
import jax.numpy as jnp
from jax import lax
from jax.experimental import pallas as pl
from jax.experimental.pallas import tpu as pltpu
from jax.experimental.pallas import tpu_sc as plsc

BATCH = 16384
EMBED = 64
NC = 2
NS = 16
NW = NC * NS
B_PER_W = BATCH // NW
CHUNK = 128
NCHUNK = B_PER_W // CHUNK
RING = 8
CH2 = 256
NCH2 = B_PER_W // CH2
RING2 = 1
SENTINEL = 0x7FFFFF


def _k1_user(su_hbm, ut3, wb_hbm, u1d_hbm, suv, colv, ucols, sbuf, ustage,
             wv, sems):
    wid = lax.axis_index("s") * NC + lax.axis_index("c")
    base = wid * B_PER_W

    pltpu.sync_copy(su_hbm.at[pl.ds(base, B_PER_W)], suv)
    pltpu.sync_copy(wb_hbm, wv)

    iot = lax.iota(jnp.int32, 16)

    for k in range(B_PER_W // 16):
        colv[pl.ds(k * 16, 16)] = jax.lax.shift_right_logical(
            suv[pl.ds(k * 16, 16)], 7)
    colv[pl.ds(B_PER_W, 16)] = jnp.full((16,), SENTINEL, jnp.int32)

    nu = jnp.int32(0)
    for k in range(B_PER_W // 16):
        cur = colv[pl.ds(k * 16, 16)]
        prev = plsc.load_gather(colv, [jnp.maximum(iot + (k * 16 - 1), 0)])
        m = cur != prev
        if k == 0:
            m = jnp.logical_or(m, iot == 0)
        ranks = plsc.cumsum(m.astype(jnp.int32))
        plsc.store_scatter(ucols, [ranks + (nu - 1)], cur, mask=m)
        nu = nu + ranks[15]

    wchunks = [wv[pl.ds(q * 16, 16)] for q in range(EMBED // 16)]
    ehq = [(iot + 16 * q) >> 3 for q in range(4)]
    elq = [(iot + 16 * q) & 7 for q in range(4)]
    eoffq = [iot + 16 * q for q in range(4)]

    def ucol(m):
        return plsc.load_gather(ucols, [jnp.full((16,), m, jnp.int32)])[0]

    def fire(m, b):
        @pl.when(m < nu)
        def _():
            c = ucol(m)
            pltpu.async_copy(
                ut3.at[:, :, pl.ds(c * CHUNK, CHUNK)], sbuf.at[b], sems.at[b])

    def drain(m, b):
        @pl.when(m < nu)
        def _():
            pltpu.make_async_copy(
                ut3.at[:, :, pl.ds(0, CHUNK)], sbuf.at[b], sems.at[b]).wait()

    def extract(m, b, carry):
        c = ucol(m)
        def cond(carry):
            jp, col_j = carry
            return jnp.logical_and(jp < B_PER_W, col_j == c)

        def wbody(carry):
            jp, _ = carry
            jpv = jnp.full((16,), jp, jnp.int32)
            su_j = plsc.load_gather(suv, [jpv])
            liv = su_j & 127
            fbase = jpv << 6
            for q in range(4):
                val = plsc.load_gather(sbuf, [jnp.full((16,), b, jnp.int32),
                                              ehq[q], elq[q], liv])
                flat = fbase + eoffq[q]
                plsc.store_scatter(
                    ustage, [jax.lax.shift_right_logical(flat, 7), flat & 127],
                    val * wchunks[q])
            jp1 = jp + 1
            col1 = plsc.load_gather(colv, [jnp.full((16,), jp1, jnp.int32)])[0]
            return (jp1, col1)

        return lax.while_loop(cond, wbody, carry)

    for r in range(RING):
        fire(jnp.int32(r), r)
    nsteps = (nu + RING - 1) // RING

    def ring_body(t, carry):
        m0 = RING * t
        for r in range(RING):
            drain(m0 + r, r)
            carry = extract(m0 + r, r, carry)
            fire(m0 + r + RING, r)
        return carry

    col0 = colv[pl.ds(0, 16)][0]
    lax.fori_loop(0, nsteps, ring_body, (jnp.int32(0), col0))

    pltpu.sync_copy(ustage, u1d_hbm.at[pl.ds(wid * (B_PER_W // 2),
                                             B_PER_W // 2)])


def _k2_join(u1d_hbm, inv_hbm, pids_hbm, product_t2, wb_hbm, out_hbm,
             iidx, pidx, iphys, pphys, ubuf, pbuf, wv, outv, sems):
    wid = lax.axis_index("s") * NC + lax.axis_index("c")
    base = wid * B_PER_W

    pltpu.sync_copy(inv_hbm.at[pl.ds(base, B_PER_W)], iidx)
    pltpu.sync_copy(pids_hbm.at[pl.ds(base, B_PER_W)], pidx)
    pltpu.sync_copy(wb_hbm, wv)

    for k in range(B_PER_W // 16):
        sl = pl.ds(k * 16, 16)
        iphys[sl] = jax.lax.shift_right_logical(iidx[sl], 1)
        pphys[sl] = jax.lax.shift_right_logical(pidx[sl], 1)

    def fire(j):
        return (
            pltpu.async_copy(
                u1d_hbm.at[iphys.at[pl.ds(j * CH2, CH2)]], ubuf,
                sems.at[0]),
            pltpu.async_copy(
                product_t2.at[pphys.at[pl.ds(j * CH2, CH2)]], pbuf,
                sems.at[1]),
        )

    lane = lax.iota(jnp.int32, 16)
    bias = wv[pl.ds(EMBED, 16)][0]

    for j in range(NCH2):
        inflight = fire(j)
        for c in inflight:
            c.wait()
        urows = ubuf
        prows = pbuf

        def body(g, carry):
            k = j * CH2 + g * 16
            rows = lane + g * 16
            ucol0 = (iidx[pl.ds(k, 16)] & 1) * EMBED
            pcol0 = (pidx[pl.ds(k, 16)] & 1) * EMBED
            acc = jnp.zeros((16,), jnp.float32)
            for e in range(EMBED):
                u = plsc.load_gather(urows, [rows, ucol0 + e])
                p = plsc.load_gather(prows, [rows, pcol0 + e])
                acc = acc + u * p
            outv[pl.ds(k, 16)] = acc + bias
            return carry

        lax.fori_loop(0, CH2 // 16, body, 0)

    pltpu.sync_copy(outv, out_hbm.at[pl.ds(base, B_PER_W)])


@jax.jit
def kernel(user_ids, product_ids, user_table, product_table, fc_w, fc_b):
    uids = user_ids.astype(jnp.int32)
    pids = product_ids.astype(jnp.int32)
    iota = lax.iota(jnp.int32, BATCH)
    su, order = lax.sort_key_val(uids, iota)
    _, inv = lax.sort_key_val(order, iota)

    ut3 = user_table.T.reshape(8, 8, user_table.shape[0])
    product_t2 = product_table.reshape(product_table.shape[0] // 2, 2 * EMBED)

    wb = jnp.zeros((128,), jnp.float32)
    wb = wb.at[:EMBED].set(fc_w[0]).at[EMBED].set(fc_b[0])

    mesh = plsc.VectorSubcoreMesh(core_axis_name="c", subcore_axis_name="s")
    params = pltpu.CompilerParams(needs_layout_passes=False)

    k1 = pl.kernel(
        _k1_user,
        out_type=jax.ShapeDtypeStruct((BATCH // 2, 2 * EMBED), jnp.float32),
        mesh=mesh, compiler_params=params,
        scratch_types=[
            pltpu.VMEM((B_PER_W,), jnp.int32),
            pltpu.VMEM((B_PER_W + 16,), jnp.int32),
            pltpu.VMEM((B_PER_W,), jnp.int32),
            pltpu.VMEM((RING, 8, 8, CHUNK), jnp.float32),
            pltpu.VMEM((B_PER_W // 2, 2 * EMBED), jnp.float32),
            pltpu.VMEM((128,), jnp.float32),
            pltpu.SemaphoreType.DMA((RING,)),
        ],
    )
    u1d = k1(su, ut3, wb)

    k2 = pl.kernel(
        _k2_join,
        out_type=jax.ShapeDtypeStruct((BATCH,), jnp.float32),
        mesh=mesh, compiler_params=params,
        scratch_types=[
            pltpu.VMEM((B_PER_W,), jnp.int32),
            pltpu.VMEM((B_PER_W,), jnp.int32),
            pltpu.VMEM((B_PER_W,), jnp.int32),
            pltpu.VMEM((B_PER_W,), jnp.int32),
            pltpu.VMEM((CH2, 2 * EMBED), jnp.float32),
            pltpu.VMEM((CH2, 2 * EMBED), jnp.float32),
            pltpu.VMEM((128,), jnp.float32),
            pltpu.VMEM((B_PER_W,), jnp.float32),
            pltpu.SemaphoreType.DMA((RING2,)),
        ],
    )
    return k2(u1d, inv, pids, product_t2, wb)
